# Initial kernel scaffold; baseline (speedup 1.0000x reference)
import jax
import jax.numpy as jnp
from jax.experimental import pallas as pl


def kernel(x, idxn, idxd, edgefeats):
    raise NotImplementedError("write your pallas kernel here")


# speedup vs baseline: 6.6733x; 6.6733x over previous
import functools

import jax
import jax.numpy as jnp
from jax import lax
from jax.experimental import pallas as pl
from jax.experimental.pallas import tpu as pltpu
from jax.experimental.pallas import tpu_sc as plsc

_N = 10000
_E = 320000
_D = 128
_DE = 16

_NPAD = 10240
_RPT = _NPAD // 16
_BLK = 128
_NBLK = _E // _BLK
_NW = 32
_NB_BASE = _NBLK // _NW
_NB_EXTRA = _NBLK - _NB_BASE * _NW

_mesh = plsc.VectorSubcoreMesh(core_axis_name="c", subcore_axis_name="s")


def _zero_rows(zrow_v, nrows, ncols):
    zero16 = jnp.zeros((16,), jnp.float32)
    for i in range(nrows):
        for j in range(ncols // 16):
            zrow_v[i, pl.ds(j * 16, 16)] = zero16


def _xsum_body(x_hbm, idxn_hbm, idxd_hbm, xpart_hbm,
               idxn_v, idxd_v, rows_v, zrow_v, xsum):
    c = lax.axis_index("c")
    s = lax.axis_index("s")
    w = s * 2 + c

    _zero_rows(zrow_v, 16, _D)

    r0 = s * _RPT
    for k in range(_RPT // 16):
        pltpu.sync_copy(zrow_v, xsum.at[pl.ds(r0 + k * 16, 16)])
    plsc.subcore_barrier()

    nb = _NB_BASE + jnp.where(w < _NB_EXTRA, 1, 0)

    def step(k, carry):
        off = pl.multiple_of((w + k * _NW) * _BLK, _BLK)
        pltpu.sync_copy(idxn_hbm.at[pl.ds(off, _BLK)], idxn_v)
        pltpu.sync_copy(idxd_hbm.at[pl.ds(off, _BLK)], idxd_v)
        pltpu.sync_copy(x_hbm.at[idxn_v], rows_v)
        pltpu.sync_copy(rows_v, xsum.at[idxd_v], add=True)
        return carry

    lax.fori_loop(0, nb, step, 0)
    plsc.subcore_barrier()

    pltpu.sync_copy(xsum.at[pl.ds(r0, _RPT)],
                    xpart_hbm.at[c].at[pl.ds(r0, _RPT)])


_xsum_call = functools.partial(
    pl.kernel,
    out_type=jax.ShapeDtypeStruct((2, _NPAD, _D), jnp.float32),
    mesh=_mesh,
    scratch_types=[
        pltpu.VMEM((_BLK,), jnp.int32),
        pltpu.VMEM((_BLK,), jnp.int32),
        pltpu.VMEM((_BLK, _D), jnp.float32),
        pltpu.VMEM((16, _D), jnp.float32),
        pltpu.VMEM_SHARED((_NPAD, _D), jnp.float32),
    ],
)(_xsum_body)


def _ef_body(idxd_hbm, eft_hbm, epart_hbm, degpart_hbm,
             idxd_v, colbuf_v, ones_v, zdeg_v, degs, *esums):
    c = lax.axis_index("c")
    s = lax.axis_index("s")
    w = s * 2 + c

    zero16 = jnp.zeros((16,), jnp.float32)
    one16 = jnp.ones((16,), jnp.float32)
    for j in range(_BLK // 16):
        zdeg_v[pl.ds(j * 16, 16)] = zero16
        ones_v[pl.ds(j * 16, 16)] = one16

    r0 = s * _RPT
    for k in range(_RPT // _BLK):
        for col in range(_DE):
            pltpu.sync_copy(zdeg_v, esums[col].at[pl.ds(r0 + k * _BLK, _BLK)])
        pltpu.sync_copy(zdeg_v, degs.at[pl.ds(r0 + k * _BLK, _BLK)])
    plsc.subcore_barrier()

    nb = _NB_BASE + jnp.where(w < _NB_EXTRA, 1, 0)

    def step(k, carry):
        off = pl.multiple_of((w + k * _NW) * _BLK, _BLK)
        pltpu.sync_copy(idxd_hbm.at[pl.ds(off, _BLK)], idxd_v)
        pltpu.sync_copy(eft_hbm.at[:, pl.ds(off, _BLK)], colbuf_v)
        for col in range(_DE):
            pltpu.sync_copy(colbuf_v.at[col], esums[col].at[idxd_v],
                            add=True)
        pltpu.sync_copy(ones_v, degs.at[idxd_v], add=True)
        return carry

    lax.fori_loop(0, nb, step, 0)
    plsc.subcore_barrier()

    for col in range(_DE):
        pltpu.sync_copy(esums[col].at[pl.ds(r0, _RPT)],
                        epart_hbm.at[c].at[col].at[pl.ds(r0, _RPT)])
    pltpu.sync_copy(degs.at[pl.ds(r0, _RPT)],
                    degpart_hbm.at[c].at[pl.ds(r0, _RPT)])


_ef_call = functools.partial(
    pl.kernel,
    out_type=(
        jax.ShapeDtypeStruct((2, _DE, _NPAD), jnp.float32),
        jax.ShapeDtypeStruct((2, _NPAD), jnp.float32),
    ),
    mesh=_mesh,
    scratch_types=[
        pltpu.VMEM((_BLK,), jnp.int32),
        pltpu.VMEM((_DE, _BLK), jnp.float32),
        pltpu.VMEM((_BLK,), jnp.float32),
        pltpu.VMEM((_BLK,), jnp.float32),
        pltpu.VMEM_SHARED((_NPAD,), jnp.float32),
    ] + [pltpu.VMEM_SHARED((_NPAD,), jnp.float32) for _ in range(_DE)],
)(_ef_body)


_BN = 1000


def _tc_body(x_ref, x0_ref, x1_ref, e0_ref, e1_ref, d0_ref, d1_ref, o_ref):
    deg = d0_ref[0] + d1_ref[0]
    inv = 1.0 / jnp.maximum(deg, 1.0)
    mask = (deg > 0.0).astype(jnp.float32)
    xs = (x0_ref[0] + x1_ref[0]) * inv
    es = (e0_ref[0] + e1_ref[0]) * inv
    xm = x_ref[...] * mask
    o_ref[...] = jnp.concatenate([xs, xm, es], axis=1)


def _tc_finalize(x, xpart, epart, degp3):
    return pl.pallas_call(
        _tc_body,
        grid=(_N // _BN,),
        in_specs=[
            pl.BlockSpec((_BN, _D), lambda r: (r, 0)),
            pl.BlockSpec((1, _BN, _D), lambda r: (0, r, 0)),
            pl.BlockSpec((1, _BN, _D), lambda r: (1, r, 0)),
            pl.BlockSpec((1, _BN, _DE), lambda r: (0, r, 0)),
            pl.BlockSpec((1, _BN, _DE), lambda r: (1, r, 0)),
            pl.BlockSpec((1, _BN, 1), lambda r: (0, r, 0)),
            pl.BlockSpec((1, _BN, 1), lambda r: (1, r, 0)),
        ],
        out_specs=pl.BlockSpec((_BN, 2 * _D + _DE), lambda r: (r, 0)),
        out_shape=jax.ShapeDtypeStruct((_N, 2 * _D + _DE), jnp.float32),
    )(x, xpart, xpart, epart, epart, degp3, degp3)


def kernel(x, idxn, idxd, edgefeats):
    xpart = _xsum_call(x, idxn, idxd)
    eft = edgefeats.T
    epart_t, degp = _ef_call(idxd, eft)
    epart = epart_t.transpose(0, 2, 1)
    degp3 = degp.reshape(2, _NPAD, 1)
    return _tc_finalize(x, xpart, epart, degp3)


# speedup vs baseline: 7.5905x; 1.1374x over previous
import functools

import jax
import jax.numpy as jnp
from jax import lax
from jax.experimental import pallas as pl
from jax.experimental.pallas import tpu as pltpu
from jax.experimental.pallas import tpu_sc as plsc

_N = 10000
_E = 320000
_D = 128
_DE = 16

_NPAD = 10240
_RPT = _NPAD // 16
_BLK = 128
_NW = 32
_NBLK = _E // _BLK
_BPW = _NBLK // _NW
_NEXTRA = _NBLK - _BPW * _NW
_BPG = 2
_NGRP = _BPW // _BPG
_K2D = 6
_K2GRP = _BPW // _K2D

_mesh = plsc.VectorSubcoreMesh(core_axis_name="c", subcore_axis_name="s")


def _xsum_body(x_hbm, idxn_hbm, idxd_hbm, xpart_hbm, *scr):
    idxn_b = scr[0:_BPG]
    idxd_b = scr[_BPG:2 * _BPG]
    rows_b = scr[2 * _BPG:3 * _BPG]
    zrow_v, xsum = scr[3 * _BPG:3 * _BPG + 2]
    gsem = scr[3 * _BPG + 2:4 * _BPG + 2]
    ssem = scr[4 * _BPG + 2:5 * _BPG + 2]

    c = lax.axis_index("c")
    s = lax.axis_index("s")
    w = s * 2 + c

    zero16 = jnp.zeros((16,), jnp.float32)
    for i in range(8):
        for j in range(_D // 16):
            zrow_v[i, pl.ds(j * 16, 16)] = zero16

    r0 = s * _RPT
    for k in range(_RPT // 8):
        pltpu.sync_copy(zrow_v, xsum.at[pl.ds(r0 + k * 8, 8)])
    plsc.subcore_barrier()

    base = (_BPW * w + jnp.minimum(w, _NEXTRA)) * _BLK

    def group(g, carry):
        goff = base + g * (_BPG * _BLK)
        for i in range(_BPG):
            off = pl.multiple_of(goff + i * _BLK, 8)
            pltpu.sync_copy(idxn_hbm.at[pl.ds(off, _BLK)], idxn_b[i])
            pltpu.sync_copy(idxd_hbm.at[pl.ds(off, _BLK)], idxd_b[i])
        gds = [pltpu.async_copy(x_hbm.at[idxn_b[i]], rows_b[i], gsem[i])
               for i in range(_BPG)]
        sds = []
        for i in range(_BPG):
            gds[i].wait()
            sds.append(pltpu.async_copy(rows_b[i], xsum.at[idxd_b[i]],
                                        ssem[i], add=True))
        for sd in sds:
            sd.wait()
        return carry

    lax.fori_loop(0, _NGRP, group, 0)

    @pl.when(w < _NEXTRA)
    def _():
        off = pl.multiple_of(base + _BPW * _BLK, _BLK)
        pltpu.sync_copy(idxn_hbm.at[pl.ds(off, _BLK)], idxn_b[0])
        pltpu.sync_copy(idxd_hbm.at[pl.ds(off, _BLK)], idxd_b[0])
        pltpu.sync_copy(x_hbm.at[idxn_b[0]], rows_b[0])
        pltpu.sync_copy(rows_b[0], xsum.at[idxd_b[0]], add=True)

    plsc.subcore_barrier()

    pltpu.sync_copy(xsum.at[pl.ds(r0, _RPT)],
                    xpart_hbm.at[c].at[pl.ds(r0, _RPT)])


_xsum_call = functools.partial(
    pl.kernel,
    out_type=jax.ShapeDtypeStruct((2, _NPAD, _D), jnp.float32),
    mesh=_mesh,
    scratch_types=(
        [pltpu.VMEM((_BLK,), jnp.int32) for _ in range(2 * _BPG)]
        + [pltpu.VMEM((_BLK, _D), jnp.float32) for _ in range(_BPG)]
        + [
            pltpu.VMEM((8, _D), jnp.float32),
            pltpu.VMEM_SHARED((_NPAD, _D), jnp.float32),
        ]
        + [pltpu.SemaphoreType.DMA for _ in range(2 * _BPG)]
    ),
)(_xsum_body)


def _ef_body(idxd_hbm, eft_hbm, epart_hbm, degpart_hbm, *scr):
    idxd_b = scr[0:_K2D]
    colb = scr[_K2D:2 * _K2D]
    ones_v, zdeg_v, degs = scr[2 * _K2D:2 * _K2D + 3]
    esem = scr[2 * _K2D + 3:3 * _K2D + 3]
    esums = scr[3 * _K2D + 3:3 * _K2D + 3 + _DE]

    c = lax.axis_index("c")
    s = lax.axis_index("s")
    w = s * 2 + c

    zero16 = jnp.zeros((16,), jnp.float32)
    one16 = jnp.ones((16,), jnp.float32)
    for j in range(_BLK // 16):
        zdeg_v[pl.ds(j * 16, 16)] = zero16
        ones_v[pl.ds(j * 16, 16)] = one16

    r0 = s * _RPT
    for k in range(_RPT // _BLK):
        for col in range(_DE):
            pltpu.sync_copy(zdeg_v, esums[col].at[pl.ds(r0 + k * _BLK, _BLK)])
        pltpu.sync_copy(zdeg_v, degs.at[pl.ds(r0 + k * _BLK, _BLK)])
    plsc.subcore_barrier()

    base = (_BPW * w + jnp.minimum(w, _NEXTRA)) * _BLK

    def group(g, carry):
        for j in range(_K2D):
            off = pl.multiple_of(base + (g * _K2D + j) * _BLK, _BLK)
            pltpu.sync_copy(idxd_hbm.at[pl.ds(off, _BLK)], idxd_b[j])
            pltpu.sync_copy(eft_hbm.at[:, pl.ds(off, _BLK)], colb[j])
        descs = []
        for j in range(_K2D):
            for col in range(_DE):
                descs.append(pltpu.async_copy(
                    colb[j].at[col], esums[col].at[idxd_b[j]], esem[j],
                    add=True))
            descs.append(pltpu.async_copy(
                ones_v, degs.at[idxd_b[j]], esem[j], add=True))
        for d in descs:
            d.wait()
        return carry

    lax.fori_loop(0, _K2GRP, group, 0)

    @pl.when(w < _NEXTRA)
    def _():
        off = pl.multiple_of(base + _BPW * _BLK, _BLK)
        pltpu.sync_copy(idxd_hbm.at[pl.ds(off, _BLK)], idxd_b[0])
        pltpu.sync_copy(eft_hbm.at[:, pl.ds(off, _BLK)], colb[0])
        for col in range(_DE):
            pltpu.sync_copy(colb[0].at[col], esums[col].at[idxd_b[0]],
                            add=True)
        pltpu.sync_copy(ones_v, degs.at[idxd_b[0]], add=True)

    plsc.subcore_barrier()

    for col in range(_DE):
        pltpu.sync_copy(esums[col].at[pl.ds(r0, _RPT)],
                        epart_hbm.at[c].at[col].at[pl.ds(r0, _RPT)])
    pltpu.sync_copy(degs.at[pl.ds(r0, _RPT)],
                    degpart_hbm.at[c].at[pl.ds(r0, _RPT)])


_ef_call = functools.partial(
    pl.kernel,
    out_type=(
        jax.ShapeDtypeStruct((2, _DE, _NPAD), jnp.float32),
        jax.ShapeDtypeStruct((2, _NPAD), jnp.float32),
    ),
    mesh=_mesh,
    scratch_types=(
        [pltpu.VMEM((_BLK,), jnp.int32) for _ in range(_K2D)]
        + [pltpu.VMEM((_DE, _BLK), jnp.float32) for _ in range(_K2D)]
        + [
            pltpu.VMEM((_BLK,), jnp.float32),
            pltpu.VMEM((_BLK,), jnp.float32),
            pltpu.VMEM_SHARED((_NPAD,), jnp.float32),
        ]
        + [pltpu.SemaphoreType.DMA for _ in range(_K2D)]
        + [pltpu.VMEM_SHARED((_NPAD,), jnp.float32) for _ in range(_DE)]
    ),
)(_ef_body)


_BN = 1000


def _tc_body(x_ref, x0_ref, x1_ref, e0_ref, e1_ref, d0_ref, d1_ref, o_ref):
    deg = d0_ref[0] + d1_ref[0]
    inv = 1.0 / jnp.maximum(deg, 1.0)
    mask = (deg > 0.0).astype(jnp.float32)
    xs = (x0_ref[0] + x1_ref[0]) * inv
    es = (e0_ref[0] + e1_ref[0]) * inv
    xm = x_ref[...] * mask
    o_ref[...] = jnp.concatenate([xs, xm, es], axis=1)


def _tc_finalize(x, xpart, epart, degp3):
    return pl.pallas_call(
        _tc_body,
        grid=(_N // _BN,),
        in_specs=[
            pl.BlockSpec((_BN, _D), lambda r: (r, 0)),
            pl.BlockSpec((1, _BN, _D), lambda r: (0, r, 0)),
            pl.BlockSpec((1, _BN, _D), lambda r: (1, r, 0)),
            pl.BlockSpec((1, _BN, _DE), lambda r: (0, r, 0)),
            pl.BlockSpec((1, _BN, _DE), lambda r: (1, r, 0)),
            pl.BlockSpec((1, _BN, 1), lambda r: (0, r, 0)),
            pl.BlockSpec((1, _BN, 1), lambda r: (1, r, 0)),
        ],
        out_specs=pl.BlockSpec((_BN, 2 * _D + _DE), lambda r: (r, 0)),
        out_shape=jax.ShapeDtypeStruct((_N, 2 * _D + _DE), jnp.float32),
    )(x, xpart, xpart, epart, epart, degp3, degp3)


def kernel(x, idxn, idxd, edgefeats):
    xpart = _xsum_call(x, idxn, idxd)
    eft = edgefeats.T
    epart_t, degp = _ef_call(idxd, eft)
    epart = epart_t.transpose(0, 2, 1)
    degp3 = degp.reshape(2, _NPAD, 1)
    return _tc_finalize(x, xpart, epart, degp3)


# speedup vs baseline: 9.4929x; 1.2506x over previous
import functools

import jax
import jax.numpy as jnp
from jax import lax
from jax.experimental import pallas as pl
from jax.experimental.pallas import tpu as pltpu
from jax.experimental.pallas import tpu_sc as plsc

_N = 10000
_E = 320000
_D = 128
_DE = 16

_NPAD = 10240
_RPT = _NPAD // 16
_BLK = 128
_NW = 32
_NBLK = _E // _BLK
_BPW = _NBLK // _NW
_NEXTRA = _NBLK - _BPW * _NW
_BPG = 2
_NGRP = _BPW // _BPG
_K2D = 6
_K2GRP = _BPW // _K2D

_mesh = plsc.VectorSubcoreMesh(core_axis_name="c", subcore_axis_name="s")


def _xsum_body(x_hbm, idxn_hbm, idxd_hbm, xpart_hbm, *scr):
    idxn_ch, idxd_ch = scr[0:2]
    idxn_v = scr[2:4]
    idxd_v = scr[4:6]
    rows_b = scr[6:8]
    zrow_v, xsum = scr[8:10]
    gsem = scr[10:12]
    ssem = scr[12:14]

    c = lax.axis_index("c")
    s = lax.axis_index("s")
    w = s * 2 + c

    zero16 = jnp.zeros((16,), jnp.float32)
    for i in range(8):
        for j in range(_D // 16):
            zrow_v[i, pl.ds(j * 16, 16)] = zero16

    r0 = s * _RPT
    for k in range(_RPT // 8):
        pltpu.sync_copy(zrow_v, xsum.at[pl.ds(r0 + k * 8, 8)])
    plsc.subcore_barrier()

    c0 = 10 * w - jnp.maximum(w - 24, 0)
    nch = jnp.where(w < 24, 10, 9)

    def stage_idx(j, b):
        for t in range(_BLK // 16):
            idxn_v[b][pl.ds(t * 16, 16)] = idxn_ch[j, pl.ds(t * 16, 16)]
            idxd_v[b][pl.ds(t * 16, 16)] = idxd_ch[j, pl.ds(t * 16, 16)]

    def chunk(q, carry):
        cblk = (c0 + q) * 8
        pltpu.sync_copy(idxn_hbm.at[pl.ds(cblk, 8)], idxn_ch)
        pltpu.sync_copy(idxd_hbm.at[pl.ds(cblk, 8)], idxd_ch)
        gds = [None, None]
        sds = [None, None]
        stage_idx(0, 0)
        gds[0] = pltpu.async_copy(x_hbm.at[idxn_v[0]], rows_b[0], gsem[0])
        stage_idx(1, 1)
        gds[1] = pltpu.async_copy(x_hbm.at[idxn_v[1]], rows_b[1], gsem[1])
        gds[0].wait()
        sds[0] = pltpu.async_copy(rows_b[0], xsum.at[idxd_v[0]], ssem[0],
                                  add=True)
        for j in range(2, 8):
            b = j % 2
            sds[b].wait()
            stage_idx(j, b)
            gds[b] = pltpu.async_copy(x_hbm.at[idxn_v[b]], rows_b[b], gsem[b])
            gds[1 - b].wait()
            sds[1 - b] = pltpu.async_copy(rows_b[1 - b],
                                          xsum.at[idxd_v[1 - b]],
                                          ssem[1 - b], add=True)
        gds[1].wait()
        sds[1] = pltpu.async_copy(rows_b[1], xsum.at[idxd_v[1]], ssem[1],
                                  add=True)
        sds[0].wait()
        sds[1].wait()
        return carry

    lax.fori_loop(0, nch, chunk, 0)

    @pl.when(w == _NW - 1)
    def _():
        pltpu.sync_copy(idxn_hbm.at[pl.ds(312 * 8, 4)],
                        idxn_ch.at[pl.ds(0, 4)])
        pltpu.sync_copy(idxd_hbm.at[pl.ds(312 * 8, 4)],
                        idxd_ch.at[pl.ds(0, 4)])
        for j in range(4):
            stage_idx(j, 0)
            pltpu.sync_copy(x_hbm.at[idxn_v[0]], rows_b[0])
            pltpu.sync_copy(rows_b[0], xsum.at[idxd_v[0]], add=True)

    plsc.subcore_barrier()

    pltpu.sync_copy(xsum.at[pl.ds(r0, _RPT)],
                    xpart_hbm.at[c].at[pl.ds(r0, _RPT)])


_xsum_call = functools.partial(
    pl.kernel,
    out_type=jax.ShapeDtypeStruct((2, _NPAD, _D), jnp.float32),
    mesh=_mesh,
    scratch_types=(
        [pltpu.VMEM((8, _BLK), jnp.int32) for _ in range(2)]
        + [pltpu.VMEM((_BLK,), jnp.int32) for _ in range(4)]
        + [pltpu.VMEM((_BLK, _D), jnp.float32) for _ in range(2)]
        + [
            pltpu.VMEM((8, _D), jnp.float32),
            pltpu.VMEM_SHARED((_NPAD, _D), jnp.float32),
        ]
        + [pltpu.SemaphoreType.DMA for _ in range(4)]
    ),
)(_xsum_body)


def _ef_body(idxd_hbm, eft_hbm, epart_hbm, degpart_hbm, *scr):
    idxd_b = scr[0:_K2D]
    colb = scr[_K2D:2 * _K2D]
    ones_v, zdeg_v, degs = scr[2 * _K2D:2 * _K2D + 3]
    esem = scr[2 * _K2D + 3:3 * _K2D + 3]
    esums = scr[3 * _K2D + 3:3 * _K2D + 3 + _DE]

    c = lax.axis_index("c")
    s = lax.axis_index("s")
    w = s * 2 + c

    zero16 = jnp.zeros((16,), jnp.float32)
    one16 = jnp.ones((16,), jnp.float32)
    for j in range(_BLK // 16):
        zdeg_v[pl.ds(j * 16, 16)] = zero16
        ones_v[pl.ds(j * 16, 16)] = one16

    r0 = s * _RPT
    for k in range(_RPT // _BLK):
        for col in range(_DE):
            pltpu.sync_copy(zdeg_v, esums[col].at[pl.ds(r0 + k * _BLK, _BLK)])
        pltpu.sync_copy(zdeg_v, degs.at[pl.ds(r0 + k * _BLK, _BLK)])
    plsc.subcore_barrier()

    base = (_BPW * w + jnp.minimum(w, _NEXTRA)) * _BLK

    def group(g, carry):
        for j in range(_K2D):
            off = pl.multiple_of(base + (g * _K2D + j) * _BLK, _BLK)
            pltpu.sync_copy(idxd_hbm.at[pl.ds(off, _BLK)], idxd_b[j])
            pltpu.sync_copy(eft_hbm.at[:, pl.ds(off, _BLK)], colb[j])
        descs = []
        for j in range(_K2D):
            for col in range(_DE):
                descs.append(pltpu.async_copy(
                    colb[j].at[col], esums[col].at[idxd_b[j]], esem[j],
                    add=True))
            descs.append(pltpu.async_copy(
                ones_v, degs.at[idxd_b[j]], esem[j], add=True))
        for d in descs:
            d.wait()
        return carry

    lax.fori_loop(0, _K2GRP, group, 0)

    @pl.when(w < _NEXTRA)
    def _():
        off = pl.multiple_of(base + _BPW * _BLK, _BLK)
        pltpu.sync_copy(idxd_hbm.at[pl.ds(off, _BLK)], idxd_b[0])
        pltpu.sync_copy(eft_hbm.at[:, pl.ds(off, _BLK)], colb[0])
        for col in range(_DE):
            pltpu.sync_copy(colb[0].at[col], esums[col].at[idxd_b[0]],
                            add=True)
        pltpu.sync_copy(ones_v, degs.at[idxd_b[0]], add=True)

    plsc.subcore_barrier()

    for col in range(_DE):
        pltpu.sync_copy(esums[col].at[pl.ds(r0, _RPT)],
                        epart_hbm.at[c].at[col].at[pl.ds(r0, _RPT)])
    pltpu.sync_copy(degs.at[pl.ds(r0, _RPT)],
                    degpart_hbm.at[c].at[pl.ds(r0, _RPT)])


_ef_call = functools.partial(
    pl.kernel,
    out_type=(
        jax.ShapeDtypeStruct((2, _DE, _NPAD), jnp.float32),
        jax.ShapeDtypeStruct((2, _NPAD), jnp.float32),
    ),
    mesh=_mesh,
    scratch_types=(
        [pltpu.VMEM((_BLK,), jnp.int32) for _ in range(_K2D)]
        + [pltpu.VMEM((_DE, _BLK), jnp.float32) for _ in range(_K2D)]
        + [
            pltpu.VMEM((_BLK,), jnp.float32),
            pltpu.VMEM((_BLK,), jnp.float32),
            pltpu.VMEM_SHARED((_NPAD,), jnp.float32),
        ]
        + [pltpu.SemaphoreType.DMA for _ in range(_K2D)]
        + [pltpu.VMEM_SHARED((_NPAD,), jnp.float32) for _ in range(_DE)]
    ),
)(_ef_body)


_BN = 1000


def _tc_body(x_ref, x0_ref, x1_ref, e0_ref, e1_ref, d0_ref, d1_ref, o_ref):
    deg = d0_ref[0] + d1_ref[0]
    inv = 1.0 / jnp.maximum(deg, 1.0)
    mask = (deg > 0.0).astype(jnp.float32)
    xs = (x0_ref[0] + x1_ref[0]) * inv
    es = (e0_ref[0] + e1_ref[0]) * inv
    xm = x_ref[...] * mask
    o_ref[...] = jnp.concatenate([xs, xm, es], axis=1)


def _tc_finalize(x, xpart, epart, degp3):
    return pl.pallas_call(
        _tc_body,
        grid=(_N // _BN,),
        in_specs=[
            pl.BlockSpec((_BN, _D), lambda r: (r, 0)),
            pl.BlockSpec((1, _BN, _D), lambda r: (0, r, 0)),
            pl.BlockSpec((1, _BN, _D), lambda r: (1, r, 0)),
            pl.BlockSpec((1, _BN, _DE), lambda r: (0, r, 0)),
            pl.BlockSpec((1, _BN, _DE), lambda r: (1, r, 0)),
            pl.BlockSpec((1, _BN, 1), lambda r: (0, r, 0)),
            pl.BlockSpec((1, _BN, 1), lambda r: (1, r, 0)),
        ],
        out_specs=pl.BlockSpec((_BN, 2 * _D + _DE), lambda r: (r, 0)),
        out_shape=jax.ShapeDtypeStruct((_N, 2 * _D + _DE), jnp.float32),
    )(x, xpart, xpart, epart, epart, degp3, degp3)


def kernel(x, idxn, idxd, edgefeats):
    idxn2 = idxn.reshape(_NBLK, _BLK)
    idxd2 = idxd.reshape(_NBLK, _BLK)
    xpart = _xsum_call(x, idxn2, idxd2)
    eft = edgefeats.T
    epart_t, degp = _ef_call(idxd, eft)
    epart = epart_t.transpose(0, 2, 1)
    degp3 = degp.reshape(2, _NPAD, 1)
    return _tc_finalize(x, xpart, epart, degp3)
